# baseline (speedup 1.0000x reference)
import functools

import jax
import jax.numpy as jnp
from jax import lax
from jax.experimental import pallas as pl
from jax.experimental.pallas import tpu as pltpu
from jax.experimental.pallas import tpu_sc as plsc

_NC = 2
_NS = 16
_NW = _NC * _NS


def _make_small_gather(num_signals, batch, num_latents, pos_dims, ori_dims):
    mesh = plsc.VectorSubcoreMesh(core_axis_name="c", subcore_axis_name="s")
    p_rows = num_latents * pos_dims
    gw_rows = num_latents

    @functools.partial(
        pl.kernel,
        mesh=mesh,
        out_type=(
            jax.ShapeDtypeStruct((2 * p_rows, batch), jnp.float32),
            jax.ShapeDtypeStruct((gw_rows, batch), jnp.float32),
        ),
        scratch_types=[
            pltpu.VMEM((batch // 128, 128), jnp.int32),
            pltpu.VMEM((batch,), jnp.float32),
            pltpu.VMEM((batch,), jnp.float32),
            pltpu.VMEM((batch,), jnp.float32),
            pltpu.SemaphoreType.DMA,
        ],
        compiler_params=pltpu.CompilerParams(use_tc_tiling_on_sc=False),
    )
    def small_kernel(idx_hbm, pp_hbm, po_hbm, gw_hbm,
                     p_out, gw_out,
                     idx_v, pp_v, po_v, gw_v, sem):
        wid = lax.axis_index("s") * _NC + lax.axis_index("c")
        n_chunks = batch // 128

        pltpu.sync_copy(idx_hbm, idx_v)

        lat = lax.div(wid, 2)
        comp = lax.rem(wid, 2)
        gw_row = wid - gw_rows
        has_gw = wid >= gw_rows

        def chunk_body(g, _):
            js = pl.ds(g * 128, 128)
            idx_c = idx_v.at[g]
            cp_pp = pltpu.async_copy(pp_hbm.at[lat, comp].at[idx_c],
                                     pp_v.at[js], sem)
            cp_po = pltpu.async_copy(po_hbm.at[lat, comp].at[idx_c],
                                     po_v.at[js], sem)

            @pl.when(has_gw)
            def _():
                pltpu.async_copy(gw_hbm.at[gw_row, 0].at[idx_c],
                                 gw_v.at[js], sem).wait()

            cp_pp.wait()
            cp_po.wait()
            return 0

        lax.fori_loop(0, n_chunks, chunk_body, 0)

        p_row = 4 * lat + comp
        pltpu.sync_copy(pp_v, p_out.at[p_row])
        pltpu.sync_copy(po_v, p_out.at[p_row + 2])

        @pl.when(has_gw)
        def _():
            pltpu.sync_copy(gw_v, gw_out.at[gw_row])

    return small_kernel


def _make_a_gather(num_signals, batch, a_rows):
    mesh = plsc.VectorSubcoreMesh(core_axis_name="c", subcore_axis_name="s")
    a_per_w = a_rows // _NW

    @functools.partial(
        pl.kernel,
        mesh=mesh,
        out_type=jax.ShapeDtypeStruct((a_rows, batch), jnp.float32),
        scratch_types=[
            pltpu.VMEM((batch // 128, 128), jnp.int32),
            pltpu.VMEM((a_per_w, batch), jnp.float32),
            pltpu.SemaphoreType.DMA,
        ],
        compiler_params=pltpu.CompilerParams(use_tc_tiling_on_sc=False),
    )
    def a_kernel(idx_hbm, a_hbm, a_out, idx_v, a_v, sem):
        wid = lax.axis_index("s") * _NC + lax.axis_index("c")
        n_chunks = batch // 128
        a_base = wid * a_per_w

        pltpu.sync_copy(idx_hbm, idx_v)

        def chunk_body(g, _):
            js = pl.ds(g * 128, 128)
            idx_c = idx_v.at[g]
            copies = [
                pltpu.async_copy(a_hbm.at[a_base + i].at[idx_c],
                                 a_v.at[i, js], sem)
                for i in range(a_per_w)
            ]
            for cp in copies:
                cp.wait()
            return 0

        lax.fori_loop(0, n_chunks, chunk_body, 0)

        pltpu.sync_copy(a_v, a_out.at[pl.ds(a_base, a_per_w)])

    return a_kernel


def kernel(idx, p_pos, p_ori, a, gaussian_window):
    num_signals, num_latents, pos_dims = p_pos.shape
    batch = idx.shape[0]
    latent_dim = a.shape[-1]
    ori_dims = p_ori.shape[-1]

    ppt = jnp.transpose(p_pos, (1, 2, 0))
    pot = jnp.transpose(p_ori, (1, 2, 0))
    gwt = jnp.transpose(gaussian_window, (1, 2, 0))
    at = jnp.transpose(a, (1, 2, 0)).reshape(num_latents * latent_dim,
                                             num_signals)

    idx2 = idx.reshape(batch // 128, 128)
    k_small = _make_small_gather(num_signals, batch, num_latents, pos_dims,
                                 ori_dims)
    k_a = _make_a_gather(num_signals, batch, at.shape[0])

    p_t, gw_t = k_small(idx2, ppt, pot, gwt)
    a_t = k_a(idx2, at)

    p = jnp.transpose(
        p_t.reshape(num_latents, pos_dims + ori_dims, batch), (2, 0, 1))
    a_g = jnp.transpose(a_t.reshape(num_latents, latent_dim, batch), (2, 0, 1))
    gw_g = jnp.transpose(gw_t.reshape(num_latents, 1, batch), (2, 0, 1))
    return (p, a_g, gw_g)


# speedup vs baseline: 1.3428x; 1.3428x over previous
import functools

import jax
import jax.numpy as jnp
from jax import lax
from jax.experimental import pallas as pl
from jax.experimental.pallas import tpu as pltpu
from jax.experimental.pallas import tpu_sc as plsc

_NC = 2
_NS = 16
_NW = _NC * _NS


def _make_sc_gather(num_signals, batch, a_cols, p_rows, gw_rows):
    b_per_w = batch // _NW
    mesh = plsc.VectorSubcoreMesh(core_axis_name="c", subcore_axis_name="s")

    @functools.partial(
        pl.kernel,
        mesh=mesh,
        out_type=(
            jax.ShapeDtypeStruct((2 * p_rows, batch), jnp.float32),
            jax.ShapeDtypeStruct((batch, a_cols), jnp.float32),
            jax.ShapeDtypeStruct((gw_rows, batch), jnp.float32),
        ),
        scratch_types=[
            pltpu.VMEM((batch // 128, 128), jnp.int32),
            pltpu.VMEM((b_per_w, a_cols), jnp.float32),
            pltpu.VMEM((batch,), jnp.float32),
            pltpu.VMEM((batch,), jnp.float32),
            pltpu.VMEM((batch,), jnp.float32),
            pltpu.SemaphoreType.DMA,
            pltpu.SemaphoreType.DMA,
        ],
        compiler_params=pltpu.CompilerParams(use_tc_tiling_on_sc=False),
    )
    def gather_kernel(idx_hbm, pp_hbm, po_hbm, a_hbm, gw_hbm,
                      p_out, a_out, gw_out,
                      idx_v, a_v, pp_v, po_v, gw_v,
                      sem_a, sem_small):
        wid = lax.axis_index("s") * _NC + lax.axis_index("c")
        n_chunks = batch // 128

        pltpu.sync_copy(idx_hbm, idx_v)

        gw_row = wid - gw_rows
        has_gw = wid >= gw_rows

        a_chunk0 = wid * (b_per_w // 128)
        a_copies = [
            pltpu.async_copy(
                a_hbm.at[idx_v.at[a_chunk0 + i]],
                a_v.at[pl.ds(128 * i, 128)], sem_a)
            for i in range(b_per_w // 128)
        ]

        def chunk_body(g, _):
            js = pl.ds(g * 128, 128)
            idx_c = idx_v.at[g]
            cp_pp = pltpu.async_copy(pp_hbm.at[wid].at[idx_c],
                                     pp_v.at[js], sem_small)
            cp_po = pltpu.async_copy(po_hbm.at[wid].at[idx_c],
                                     po_v.at[js], sem_small)

            @pl.when(has_gw)
            def _():
                pltpu.async_copy(gw_hbm.at[gw_row].at[idx_c],
                                 gw_v.at[js], sem_small).wait()

            cp_pp.wait()
            cp_po.wait()
            return 0

        lax.fori_loop(0, n_chunks, chunk_body, 0)

        p_row = 2 * wid - lax.rem(wid, 2)
        pltpu.sync_copy(pp_v, p_out.at[p_row])
        pltpu.sync_copy(po_v, p_out.at[p_row + 2])

        @pl.when(has_gw)
        def _():
            pltpu.sync_copy(gw_v, gw_out.at[gw_row])

        for cp in a_copies:
            cp.wait()
        pltpu.sync_copy(a_v, a_out.at[pl.ds(wid * b_per_w, b_per_w)])

    return gather_kernel


def kernel(idx, p_pos, p_ori, a, gaussian_window):
    num_signals, num_latents, pos_dims = p_pos.shape
    batch = idx.shape[0]
    latent_dim = a.shape[-1]
    ori_dims = p_ori.shape[-1]

    ppt = jnp.transpose(p_pos, (1, 2, 0)).reshape(num_latents * pos_dims,
                                                  num_signals)
    pot = jnp.transpose(p_ori, (1, 2, 0)).reshape(num_latents * ori_dims,
                                                  num_signals)
    gwt = jnp.transpose(gaussian_window, (1, 2, 0)).reshape(num_latents,
                                                            num_signals)
    a_sm = a.reshape(num_signals, num_latents * latent_dim)

    fn = _make_sc_gather(num_signals, batch, a_sm.shape[1], ppt.shape[0],
                         gwt.shape[0])
    p_t, a_r, gw_t = fn(idx.reshape(batch // 128, 128), ppt, pot, a_sm, gwt)

    p = jnp.transpose(
        p_t.reshape(num_latents, pos_dims + ori_dims, batch), (2, 0, 1))
    a_g = a_r.reshape(batch, num_latents, latent_dim)
    gw_g = jnp.transpose(gw_t.reshape(num_latents, 1, batch), (2, 0, 1))
    return (p, a_g, gw_g)
